# baseline (speedup 1.0000x reference)
import functools

import jax
import jax.numpy as jnp
from jax import lax
from jax.experimental import pallas as pl
from jax.experimental.pallas import tpu as pltpu
from jax.experimental.pallas import tpu_sc as plsc

_N = 10000
_E = 320000
_D = 128
_L = 3
_VOCAB = 32

_NC = 2
_NS = 16
_NW = _NC * _NS
_EPW = _E // _NW
_CHUNK = 80
_NCHUNK = _EPW // _CHUNK
_RING = 88
_RPT = 632

_BLK = 2000
_NBLK = _N // _BLK


def _sc_agg_body(x_hbm, src_hbm, meta_hbm, scat_hbm, zeros_hbm, out_hbm,
                 srcidx_v, meta_v, rows_v, ring_v, scat_v, aggsh, sem):
    c = lax.axis_index("c")
    s = lax.axis_index("s")
    w = c * _NS + s

    stripe = pl.multiple_of(
        jnp.minimum(s * _RPT, _N + 8 - _RPT).astype(jnp.int32), 8)
    pltpu.sync_copy(zeros_hbm.at[pl.ds(stripe, _RPT)],
                    aggsh.at[pl.ds(stripe, _RPT)])
    plsc.subcore_barrier()

    ebase = w * _EPW

    def chunk_body(g, acc):
        base = ebase + g * _CHUNK
        pltpu.sync_copy(src_hbm.at[pl.ds(base, _CHUNK)], srcidx_v)
        pltpu.sync_copy(meta_hbm.at[pl.ds(base, _CHUNK)],
                        meta_v.at[pl.ds(0, _CHUNK)])
        pltpu.sync_copy(scat_hbm.at[pl.ds((w * _NCHUNK + g) * _RING, _RING)],
                        scat_v)
        pltpu.async_copy(x_hbm.at[srcidx_v], rows_v, sem).wait()

        def edge_body(k, acc):
            mv = meta_v[pl.ds(k, 16)][0]
            slot = mv >> 1
            m = jnp.full((16,), (1 - (mv & 1)).astype(jnp.float32),
                         jnp.float32)
            new = []
            for ci in range(8):
                row = rows_v[k, pl.ds(16 * ci, 16)]
                a = acc[ci] * m + row
                ring_v[slot, pl.ds(16 * ci, 16)] = a
                new.append(a)
            return tuple(new)

        acc = lax.fori_loop(0, _CHUNK, edge_body, acc)
        pltpu.sync_copy(ring_v, aggsh.at[scat_v], add=True)
        return acc

    acc0 = tuple(jnp.zeros((16,), jnp.float32) for _ in range(8))
    lax.fori_loop(0, _NCHUNK, chunk_body, acc0)
    plsc.subcore_barrier()

    pltpu.sync_copy(aggsh.at[pl.ds(stripe, _RPT)],
                    out_hbm.at[c, pl.ds(stripe, _RPT)])


@functools.lru_cache(maxsize=None)
def _sc_agg_kernel():
    return functools.partial(
        pl.kernel,
        out_type=jax.ShapeDtypeStruct((_NC, _N + 8, _D), jnp.float32),
        mesh=plsc.VectorSubcoreMesh(core_axis_name="c", subcore_axis_name="s"),
        scratch_types=[
            pltpu.VMEM((_CHUNK,), jnp.int32),
            pltpu.VMEM((_CHUNK + 16,), jnp.int32),
            pltpu.VMEM((_CHUNK, _D), jnp.float32),
            pltpu.VMEM((_RING, _D), jnp.float32),
            pltpu.VMEM((_RING,), jnp.int32),
            pltpu.VMEM_SHARED((_N + 8, _D), jnp.float32),
            pltpu.SemaphoreType.DMA,
        ],
    )(_sc_agg_body)


def _sc_agg(x, src_sorted, meta, scat, zeros):
    return _sc_agg_kernel()(x, src_sorted, meta, scat, zeros)


def _edge_metadata(src, dst):
    order = jnp.argsort(dst, stable=True)
    so = jnp.take(src, order)
    do = jnp.take(dst, order)
    e = jnp.arange(_E, dtype=jnp.int32)
    prev = jnp.concatenate([jnp.full((1,), -1, jnp.int32), do[:-1]])
    flag = ((do != prev) | (e % _EPW == 0)).astype(jnp.int32)
    run_id = jnp.cumsum(flag) - 1
    rs = run_id.reshape(_E // _CHUNK, _CHUNK)
    slot = (rs - rs[:, :1]).astype(jnp.int32)
    meta = (slot.reshape(_E) * 2 + flag).astype(jnp.int32)

    run_node = jnp.zeros((_E,), jnp.int32).at[run_id].set(do)
    first_run = rs[:, 0]
    smax = rs[:, -1] - rs[:, 0]
    flag_ext = jnp.concatenate([flag, jnp.ones((1,), jnp.int32)])
    chunk_ends = (jnp.arange(_E // _CHUNK, dtype=jnp.int32) + 1) * _CHUNK
    nclosed = smax + flag_ext[chunk_ends]
    r = jnp.arange(_RING, dtype=jnp.int32)
    slots_r = first_run[:, None] + r[None, :]
    valid = r[None, :] < nclosed[:, None]
    scat = jnp.where(valid, run_node[jnp.clip(slots_r, 0, _E - 1)], _N)
    return so, meta, scat.reshape(-1).astype(jnp.int32)


_HI = lax.Precision.HIGHEST


def _emb_body(depth_ref, demb_ref, out_ref):
    oh = (depth_ref[...] == lax.broadcasted_iota(
        jnp.int32, (_BLK, _VOCAB), 1)).astype(jnp.float32)
    out_ref[...] = jnp.dot(oh, demb_ref[...],
                           preferred_element_type=jnp.float32, precision=_HI)


def _enc_body(feat_ref, wenc_ref, benc_ref, emb_ref, out_ref):
    x = jnp.dot(feat_ref[...], wenc_ref[...],
                preferred_element_type=jnp.float32)
    out_ref[...] = x + benc_ref[...] + emb_ref[...]


def _colsum8(v, blk):
    acc = jnp.zeros((8, _D), jnp.float32)

    def body(i, acc):
        return acc + v[pl.ds(i * 8, 8), :]

    return lax.fori_loop(0, blk // 8, body, acc)


def _tree8(a):
    t = a[0:4] + a[4:8]
    t = t[0:2] + t[2:4]
    return t[0:1] + t[1:2]


def _mlp_body(x_ref, a0_ref, a1_ref, w1, b1, w2, b2, w3, b3,
              h_ref, sum_ref):
    i = pl.program_id(0)
    t = x_ref[...] + (a0_ref[...] + a1_ref[...])
    h = jnp.maximum(jnp.dot(t, w1[...],
                            preferred_element_type=jnp.float32) + b1[...], 0.0)
    h = jnp.maximum(jnp.dot(h, w2[...],
                            preferred_element_type=jnp.float32) + b2[...], 0.0)
    h = jnp.maximum(jnp.dot(h, w3[...],
                            preferred_element_type=jnp.float32) + b3[...], 0.0)
    h_ref[...] = h

    def body(j, acc):
        return acc + h_ref[pl.ds(j * 8, 8), :]

    part = lax.fori_loop(0, _BLK // 8, body, jnp.zeros((8, _D), jnp.float32))

    @pl.when(i == 0)
    def _():
        sum_ref[...] = part

    @pl.when(i != 0)
    def _():
        sum_ref[...] = sum_ref[...] + part


def _var_body(h_ref, sum_ref, var_ref):
    i = pl.program_id(0)
    mean = _tree8(sum_ref[...]) / jnp.float32(_N)

    def body(j, acc):
        blk = h_ref[pl.ds(j * 8, 8), :] - mean
        return acc + blk * blk

    part = lax.fori_loop(0, _BLK // 8, body, jnp.zeros((8, _D), jnp.float32))

    @pl.when(i == 0)
    def _():
        var_ref[...] = part

    @pl.when(i != 0)
    def _():
        var_ref[...] = var_ref[...] + part


def _bn_body(h_ref, sum_ref, var_ref, g_ref, b_ref, out_ref, *, apply_relu):
    mean = _tree8(sum_ref[...]) / jnp.float32(_N)
    var = _tree8(var_ref[...]) / jnp.float32(_N)
    inv = g_ref[...] * lax.rsqrt(var + 1e-5)
    o = (h_ref[...] - mean) * inv + b_ref[...]
    if apply_relu:
        o = jnp.maximum(o, 0.0)
    out_ref[...] = o


_row_spec = pl.BlockSpec((_BLK, _D), lambda i: (i, 0))
_acc_spec = pl.BlockSpec((8, _D), lambda i: (0, 0))


def _enc_call(feat, depth2d, wenc, benc2d, demb):
    emb = pl.pallas_call(
        _emb_body,
        grid=(_NBLK,),
        in_specs=[
            pl.BlockSpec((_BLK, 1), lambda i: (i, 0)),
            pl.BlockSpec((_VOCAB, _D), lambda i: (0, 0)),
        ],
        out_specs=_row_spec,
        out_shape=jax.ShapeDtypeStruct((_N, _D), jnp.float32),
    )(depth2d, demb)
    return pl.pallas_call(
        _enc_body,
        grid=(_NBLK,),
        in_specs=[
            _row_spec,
            pl.BlockSpec((_D, _D), lambda i: (0, 0)),
            pl.BlockSpec((1, _D), lambda i: (0, 0)),
            _row_spec,
        ],
        out_specs=_row_spec,
        out_shape=jax.ShapeDtypeStruct((_N, _D), jnp.float32),
    )(feat, wenc, benc2d, emb)


def _mlp_call(x, a0, a1, w1, b1, w2, b2, w3, b3):
    wspec = pl.BlockSpec((_D, _D), lambda i: (0, 0))
    bspec = pl.BlockSpec((1, _D), lambda i: (0, 0))
    return pl.pallas_call(
        _mlp_body,
        grid=(_NBLK,),
        in_specs=[_row_spec, _row_spec, _row_spec,
                  wspec, bspec, wspec, bspec, wspec, bspec],
        out_specs=[_row_spec, _acc_spec],
        out_shape=[jax.ShapeDtypeStruct((_N, _D), jnp.float32),
                   jax.ShapeDtypeStruct((8, _D), jnp.float32)],
    )(x, a0, a1, w1, b1, w2, b2, w3, b3)


def _var_call(h, hsum):
    return pl.pallas_call(
        _var_body,
        grid=(_NBLK,),
        in_specs=[_row_spec, _acc_spec],
        out_specs=_acc_spec,
        out_shape=jax.ShapeDtypeStruct((8, _D), jnp.float32),
    )(h, hsum)


def _bn_call(h, hsum, hvar, g, b, apply_relu):
    return pl.pallas_call(
        functools.partial(_bn_body, apply_relu=apply_relu),
        grid=(_NBLK,),
        in_specs=[_row_spec, _acc_spec, _acc_spec,
                  pl.BlockSpec((1, _D), lambda i: (0, 0)),
                  pl.BlockSpec((1, _D), lambda i: (0, 0))],
        out_specs=_row_spec,
        out_shape=jax.ShapeDtypeStruct((_N, _D), jnp.float32),
    )(h, hsum, hvar, g, b)


def kernel(feat, depth, edge_index, W_enc, b_enc, depth_emb,
           W1, b1, W2, b2, W3, b3, gamma, beta):
    src = edge_index[0]
    dst = edge_index[1]
    so, meta, scat = _edge_metadata(src, dst)
    zeros = jnp.zeros((_N + 8, _D), jnp.float32)

    x = _enc_call(feat, depth.reshape(_N, 1), W_enc,
                  b_enc.reshape(1, _D), depth_emb)

    b1r = b1.reshape(1, _D)
    b2r = b2.reshape(1, _D)
    b3r = b3.reshape(1, _D)
    for l in range(_L):
        agg = _sc_agg(x, so, meta, scat, zeros)
        h, hsum = _mlp_call(x, agg[0, :_N], agg[1, :_N],
                            W1, b1r, W2, b2r, W3, b3r)
        hvar = _var_call(h, hsum)
        x = _bn_call(h, hsum, hvar, gamma[l].reshape(1, _D),
                     beta[l].reshape(1, _D), apply_relu=(l < _L - 1))
    return x


# speedup vs baseline: 1.0064x; 1.0064x over previous
import functools

import jax
import jax.numpy as jnp
from jax import lax
from jax.experimental import pallas as pl
from jax.experimental.pallas import tpu as pltpu
from jax.experimental.pallas import tpu_sc as plsc

_N = 10000
_E = 320000
_D = 128
_L = 3
_VOCAB = 32

_NC = 2
_NS = 16
_NW = _NC * _NS
_EPW = _E // _NW
_CHUNK = 80
_NCHUNK = _EPW // _CHUNK
_RING = 88
_RPT = 632

_BLK = 2000
_NBLK = _N // _BLK


def _sc_agg_body(x_hbm, src_hbm, meta_hbm, scat_hbm, zeros_hbm, out_hbm,
                 srcidx_v, meta_v, rows_v, ring_v, scat_v, aggsh, sem):
    c = lax.axis_index("c")
    s = lax.axis_index("s")
    w = c * _NS + s

    stripe = pl.multiple_of(
        jnp.minimum(s * _RPT, _N + 8 - _RPT).astype(jnp.int32), 8)
    pltpu.sync_copy(zeros_hbm.at[pl.ds(stripe, _RPT)],
                    aggsh.at[pl.ds(stripe, _RPT)])
    plsc.subcore_barrier()

    ebase = w * _EPW

    def chunk_body(g, acc):
        base = ebase + g * _CHUNK
        pltpu.sync_copy(src_hbm.at[pl.ds(base, _CHUNK)], srcidx_v)
        pltpu.sync_copy(meta_hbm.at[pl.ds(base, _CHUNK)],
                        meta_v.at[pl.ds(0, _CHUNK)])
        pltpu.sync_copy(scat_hbm.at[pl.ds((w * _NCHUNK + g) * _RING, _RING)],
                        scat_v)
        pltpu.async_copy(x_hbm.at[srcidx_v], rows_v, sem).wait()

        def edge_body(k, acc):
            mv = meta_v[pl.ds(k, 16)][0]
            slot = mv >> 1
            m = jnp.full((16,), (1 - (mv & 1)).astype(jnp.float32),
                         jnp.float32)
            new = []
            for ci in range(8):
                row = rows_v[k, pl.ds(16 * ci, 16)]
                a = acc[ci] * m + row
                ring_v[slot, pl.ds(16 * ci, 16)] = a
                new.append(a)
            return tuple(new)

        acc = lax.fori_loop(0, _CHUNK, edge_body, acc, unroll=8)
        pltpu.sync_copy(ring_v, aggsh.at[scat_v], add=True)
        return acc

    acc0 = tuple(jnp.zeros((16,), jnp.float32) for _ in range(8))
    lax.fori_loop(0, _NCHUNK, chunk_body, acc0)
    plsc.subcore_barrier()

    pltpu.sync_copy(aggsh.at[pl.ds(stripe, _RPT)],
                    out_hbm.at[c, pl.ds(stripe, _RPT)])


@functools.lru_cache(maxsize=None)
def _sc_agg_kernel():
    return functools.partial(
        pl.kernel,
        out_type=jax.ShapeDtypeStruct((_NC, _N + 8, _D), jnp.float32),
        mesh=plsc.VectorSubcoreMesh(core_axis_name="c", subcore_axis_name="s"),
        scratch_types=[
            pltpu.VMEM((_CHUNK,), jnp.int32),
            pltpu.VMEM((_CHUNK + 16,), jnp.int32),
            pltpu.VMEM((_CHUNK, _D), jnp.float32),
            pltpu.VMEM((_RING, _D), jnp.float32),
            pltpu.VMEM((_RING,), jnp.int32),
            pltpu.VMEM_SHARED((_N + 8, _D), jnp.float32),
            pltpu.SemaphoreType.DMA,
        ],
    )(_sc_agg_body)


def _sc_agg(x, src_sorted, meta, scat, zeros):
    return _sc_agg_kernel()(x, src_sorted, meta, scat, zeros)


def _edge_metadata(src, dst):
    order = jnp.argsort(dst, stable=True)
    so = jnp.take(src, order)
    do = jnp.take(dst, order)
    e = jnp.arange(_E, dtype=jnp.int32)
    prev = jnp.concatenate([jnp.full((1,), -1, jnp.int32), do[:-1]])
    flag = ((do != prev) | (e % _EPW == 0)).astype(jnp.int32)
    run_id = jnp.cumsum(flag) - 1
    rs = run_id.reshape(_E // _CHUNK, _CHUNK)
    slot = (rs - rs[:, :1]).astype(jnp.int32)
    meta = (slot.reshape(_E) * 2 + flag).astype(jnp.int32)

    run_node = jnp.zeros((_E,), jnp.int32).at[run_id].set(do)
    first_run = rs[:, 0]
    smax = rs[:, -1] - rs[:, 0]
    flag_ext = jnp.concatenate([flag, jnp.ones((1,), jnp.int32)])
    chunk_ends = (jnp.arange(_E // _CHUNK, dtype=jnp.int32) + 1) * _CHUNK
    nclosed = smax + flag_ext[chunk_ends]
    r = jnp.arange(_RING, dtype=jnp.int32)
    slots_r = first_run[:, None] + r[None, :]
    valid = r[None, :] < nclosed[:, None]
    scat = jnp.where(valid, run_node[jnp.clip(slots_r, 0, _E - 1)], _N)
    return so, meta, scat.reshape(-1).astype(jnp.int32)


_HI = lax.Precision.HIGHEST


def _emb_body(depth_ref, demb_ref, out_ref):
    oh = (depth_ref[...] == lax.broadcasted_iota(
        jnp.int32, (_BLK, _VOCAB), 1)).astype(jnp.float32)
    out_ref[...] = jnp.dot(oh, demb_ref[...],
                           preferred_element_type=jnp.float32, precision=_HI)


def _enc_body(feat_ref, wenc_ref, benc_ref, emb_ref, out_ref):
    x = jnp.dot(feat_ref[...], wenc_ref[...],
                preferred_element_type=jnp.float32)
    out_ref[...] = x + benc_ref[...] + emb_ref[...]


def _colsum8(v, blk):
    acc = jnp.zeros((8, _D), jnp.float32)

    def body(i, acc):
        return acc + v[pl.ds(i * 8, 8), :]

    return lax.fori_loop(0, blk // 8, body, acc)


def _tree8(a):
    t = a[0:4] + a[4:8]
    t = t[0:2] + t[2:4]
    return t[0:1] + t[1:2]


def _mlp_body(x_ref, a0_ref, a1_ref, w1, b1, w2, b2, w3, b3,
              h_ref, sum_ref):
    i = pl.program_id(0)
    t = x_ref[...] + (a0_ref[...] + a1_ref[...])
    h = jnp.maximum(jnp.dot(t, w1[...],
                            preferred_element_type=jnp.float32) + b1[...], 0.0)
    h = jnp.maximum(jnp.dot(h, w2[...],
                            preferred_element_type=jnp.float32) + b2[...], 0.0)
    h = jnp.maximum(jnp.dot(h, w3[...],
                            preferred_element_type=jnp.float32) + b3[...], 0.0)
    h_ref[...] = h

    def body(j, acc):
        return acc + h_ref[pl.ds(j * 8, 8), :]

    part = lax.fori_loop(0, _BLK // 8, body, jnp.zeros((8, _D), jnp.float32))

    @pl.when(i == 0)
    def _():
        sum_ref[...] = part

    @pl.when(i != 0)
    def _():
        sum_ref[...] = sum_ref[...] + part


def _var_body(h_ref, sum_ref, var_ref):
    i = pl.program_id(0)
    mean = _tree8(sum_ref[...]) / jnp.float32(_N)

    def body(j, acc):
        blk = h_ref[pl.ds(j * 8, 8), :] - mean
        return acc + blk * blk

    part = lax.fori_loop(0, _BLK // 8, body, jnp.zeros((8, _D), jnp.float32))

    @pl.when(i == 0)
    def _():
        var_ref[...] = part

    @pl.when(i != 0)
    def _():
        var_ref[...] = var_ref[...] + part


def _bn_body(h_ref, sum_ref, var_ref, g_ref, b_ref, out_ref, *, apply_relu):
    mean = _tree8(sum_ref[...]) / jnp.float32(_N)
    var = _tree8(var_ref[...]) / jnp.float32(_N)
    inv = g_ref[...] * lax.rsqrt(var + 1e-5)
    o = (h_ref[...] - mean) * inv + b_ref[...]
    if apply_relu:
        o = jnp.maximum(o, 0.0)
    out_ref[...] = o


_row_spec = pl.BlockSpec((_BLK, _D), lambda i: (i, 0))
_acc_spec = pl.BlockSpec((8, _D), lambda i: (0, 0))


def _enc_call(feat, depth2d, wenc, benc2d, demb):
    emb = pl.pallas_call(
        _emb_body,
        grid=(_NBLK,),
        in_specs=[
            pl.BlockSpec((_BLK, 1), lambda i: (i, 0)),
            pl.BlockSpec((_VOCAB, _D), lambda i: (0, 0)),
        ],
        out_specs=_row_spec,
        out_shape=jax.ShapeDtypeStruct((_N, _D), jnp.float32),
    )(depth2d, demb)
    return pl.pallas_call(
        _enc_body,
        grid=(_NBLK,),
        in_specs=[
            _row_spec,
            pl.BlockSpec((_D, _D), lambda i: (0, 0)),
            pl.BlockSpec((1, _D), lambda i: (0, 0)),
            _row_spec,
        ],
        out_specs=_row_spec,
        out_shape=jax.ShapeDtypeStruct((_N, _D), jnp.float32),
    )(feat, wenc, benc2d, emb)


def _mlp_call(x, a0, a1, w1, b1, w2, b2, w3, b3):
    wspec = pl.BlockSpec((_D, _D), lambda i: (0, 0))
    bspec = pl.BlockSpec((1, _D), lambda i: (0, 0))
    return pl.pallas_call(
        _mlp_body,
        grid=(_NBLK,),
        in_specs=[_row_spec, _row_spec, _row_spec,
                  wspec, bspec, wspec, bspec, wspec, bspec],
        out_specs=[_row_spec, _acc_spec],
        out_shape=[jax.ShapeDtypeStruct((_N, _D), jnp.float32),
                   jax.ShapeDtypeStruct((8, _D), jnp.float32)],
    )(x, a0, a1, w1, b1, w2, b2, w3, b3)


def _var_call(h, hsum):
    return pl.pallas_call(
        _var_body,
        grid=(_NBLK,),
        in_specs=[_row_spec, _acc_spec],
        out_specs=_acc_spec,
        out_shape=jax.ShapeDtypeStruct((8, _D), jnp.float32),
    )(h, hsum)


def _bn_call(h, hsum, hvar, g, b, apply_relu):
    return pl.pallas_call(
        functools.partial(_bn_body, apply_relu=apply_relu),
        grid=(_NBLK,),
        in_specs=[_row_spec, _acc_spec, _acc_spec,
                  pl.BlockSpec((1, _D), lambda i: (0, 0)),
                  pl.BlockSpec((1, _D), lambda i: (0, 0))],
        out_specs=_row_spec,
        out_shape=jax.ShapeDtypeStruct((_N, _D), jnp.float32),
    )(h, hsum, hvar, g, b)


def kernel(feat, depth, edge_index, W_enc, b_enc, depth_emb,
           W1, b1, W2, b2, W3, b3, gamma, beta):
    src = edge_index[0]
    dst = edge_index[1]
    so, meta, scat = _edge_metadata(src, dst)
    zeros = jnp.zeros((_N + 8, _D), jnp.float32)

    x = _enc_call(feat, depth.reshape(_N, 1), W_enc,
                  b_enc.reshape(1, _D), depth_emb)

    b1r = b1.reshape(1, _D)
    b2r = b2.reshape(1, _D)
    b3r = b3.reshape(1, _D)
    for l in range(_L):
        agg = _sc_agg(x, so, meta, scat, zeros)
        h, hsum = _mlp_call(x, agg[0, :_N], agg[1, :_N],
                            W1, b1r, W2, b2r, W3, b3r)
        hvar = _var_call(h, hsum)
        x = _bn_call(h, hsum, hvar, gamma[l].reshape(1, _D),
                     beta[l].reshape(1, _D), apply_relu=(l < _L - 1))
    return x


# speedup vs baseline: 1.1904x; 1.1828x over previous
import functools

import jax
import jax.numpy as jnp
from jax import lax
from jax.experimental import pallas as pl
from jax.experimental.pallas import tpu as pltpu
from jax.experimental.pallas import tpu_sc as plsc

_N = 10000
_E = 320000
_D = 128
_L = 3
_VOCAB = 32

_NC = 2
_NS = 16
_NW = _NC * _NS
_EPW = _E // _NW
_CHUNK = 80
_NCHUNK = _EPW // _CHUNK
_NDUMP = 128
_NOUT = _N + _NDUMP + 2 * _NS
_RPT = 640

_BLK = 2000
_NBLK = _N // _BLK


def _sc_agg_body(x_hbm, src_hbm, mflag_hbm, scat_hbm, fin_hbm, zeros_hbm,
                 out0_hbm, out1_hbm,
                 srcidx_v, mflag_v, scat_v, idx16_v, rows_v, ring_v, sem):
    c = lax.axis_index("c")
    s = lax.axis_index("s")
    w = c * _NS + s

    stripe = pl.multiple_of(
        jnp.minimum(s * _RPT, _NOUT - _RPT).astype(jnp.int32), 8)

    @pl.when(c == 0)
    def _():
        pltpu.sync_copy(zeros_hbm.at[pl.ds(stripe, _RPT)],
                        out0_hbm.at[pl.ds(stripe, _RPT)])

    @pl.when(c == 1)
    def _():
        pltpu.sync_copy(zeros_hbm.at[pl.ds(stripe, _RPT)],
                        out1_hbm.at[pl.ds(stripe, _RPT)])

    plsc.subcore_barrier()

    ebase = w * _EPW

    def chunk_body(g, acc):
        base = ebase + g * _CHUNK
        pltpu.sync_copy(src_hbm.at[pl.ds(base, _CHUNK)], srcidx_v)
        pltpu.sync_copy(mflag_hbm.at[pl.ds(base, _CHUNK)],
                        mflag_v.at[pl.ds(0, _CHUNK)])
        pltpu.sync_copy(scat_hbm.at[pl.ds(base, _CHUNK)], scat_v)
        pltpu.async_copy(x_hbm.at[srcidx_v], rows_v, sem).wait()

        def edge_body(k, acc):
            m = jnp.full((16,), mflag_v[pl.ds(k, 16)][0], jnp.float32)
            new = []
            for ci in range(8):
                row = rows_v[k, pl.ds(16 * ci, 16)]
                a = acc[ci] * m + row
                ring_v[k, pl.ds(16 * ci, 16)] = a
                new.append(a)
            return tuple(new)

        acc = lax.fori_loop(0, _CHUNK, edge_body, acc, unroll=8)

        @pl.when(c == 0)
        def _():
            pltpu.sync_copy(ring_v, out0_hbm.at[scat_v])

        @pl.when(c == 1)
        def _():
            pltpu.sync_copy(ring_v, out1_hbm.at[scat_v])

        return acc

    acc0 = tuple(jnp.zeros((16,), jnp.float32) for _ in range(8))
    acc = lax.fori_loop(0, _NCHUNK, chunk_body, acc0)

    for ci in range(8):
        ring_v[0, pl.ds(16 * ci, 16)] = acc[ci]
    pltpu.sync_copy(fin_hbm.at[pl.ds(w * 16, 16)], idx16_v)

    @pl.when(c == 0)
    def _():
        pltpu.sync_copy(ring_v.at[pl.ds(0, 16)], out0_hbm.at[idx16_v])

    @pl.when(c == 1)
    def _():
        pltpu.sync_copy(ring_v.at[pl.ds(0, 16)], out1_hbm.at[idx16_v])


@functools.lru_cache(maxsize=None)
def _sc_agg_kernel():
    return functools.partial(
        pl.kernel,
        out_type=[jax.ShapeDtypeStruct((_NOUT, _D), jnp.float32),
                  jax.ShapeDtypeStruct((_NOUT, _D), jnp.float32)],
        mesh=plsc.VectorSubcoreMesh(core_axis_name="c", subcore_axis_name="s"),
        scratch_types=[
            pltpu.VMEM((_CHUNK,), jnp.int32),
            pltpu.VMEM((_CHUNK + 16,), jnp.float32),
            pltpu.VMEM((_CHUNK,), jnp.int32),
            pltpu.VMEM((16,), jnp.int32),
            pltpu.VMEM((_CHUNK, _D), jnp.float32),
            pltpu.VMEM((_CHUNK, _D), jnp.float32),
            pltpu.SemaphoreType.DMA,
        ],
    )(_sc_agg_body)


def _sc_agg(x, src_sorted, mflag, scat, fin, zeros):
    return _sc_agg_kernel()(x, src_sorted, mflag, scat, fin, zeros)


def _edge_metadata(src, dst):
    order = jnp.argsort(dst, stable=True)
    so = jnp.take(src, order)
    do = jnp.take(dst, order)
    e = jnp.arange(_E, dtype=jnp.int32)
    prev = jnp.concatenate([jnp.full((1,), -1, jnp.int32), do[:-1]])
    flag = ((do != prev) | (e % _EPW == 0)).astype(jnp.int32)
    mflag = (1 - flag).astype(jnp.float32)
    run_id = jnp.cumsum(flag) - 1
    rid2 = run_id.reshape(_NW, _EPW)
    is_first = (rid2 == rid2[:, :1]).reshape(_E)
    is_last = (rid2 == rid2[:, -1:]).reshape(_E)
    closes = jnp.concatenate([flag[1:], jnp.ones((1,), jnp.int32)]) == 1
    s_local = ((e // _EPW) % _NS).astype(jnp.int32)
    dump = (_N + (e % _NDUMP)).astype(jnp.int32)
    side_first = _N + _NDUMP + 2 * s_local
    scat = jnp.where(closes & ~is_last,
                     jnp.where(is_first, side_first, do), dump)

    lanes = jnp.arange(16, dtype=jnp.int32)
    tiles = jnp.arange(_NW, dtype=jnp.int32)
    fin = jnp.where(lanes[None, :] == 0,
                    (_N + _NDUMP + 2 * (tiles % _NS) + 1)[:, None],
                    _N + lanes[None, :] * 8).astype(jnp.int32)

    d2 = do.reshape(_NW, _EPW)
    nf = d2[:, 0].reshape(_NC, _NS)
    nl = d2[:, -1].reshape(_NC, _NS)
    nodecol = jnp.stack([nf, nl], axis=2).reshape(2 * _NW)
    selT = (jnp.arange(_N, dtype=jnp.int32)[:, None] ==
            nodecol[None, :]).astype(jnp.float32)
    return so, mflag, scat.astype(jnp.int32), fin.reshape(-1), selT


_HI = lax.Precision.HIGHEST


def _emb_body(depth_ref, demb_ref, out_ref):
    oh = (depth_ref[...] == lax.broadcasted_iota(
        jnp.int32, (_BLK, _VOCAB), 1)).astype(jnp.float32)
    out_ref[...] = jnp.dot(oh, demb_ref[...],
                           preferred_element_type=jnp.float32, precision=_HI)


def _enc_body(feat_ref, wenc_ref, benc_ref, emb_ref, out_ref):
    x = jnp.dot(feat_ref[...], wenc_ref[...],
                preferred_element_type=jnp.float32)
    out_ref[...] = x + benc_ref[...] + emb_ref[...]


def _colsum8(v, blk):
    acc = jnp.zeros((8, _D), jnp.float32)

    def body(i, acc):
        return acc + v[pl.ds(i * 8, 8), :]

    return lax.fori_loop(0, blk // 8, body, acc)


def _tree8(a):
    t = a[0:4] + a[4:8]
    t = t[0:2] + t[2:4]
    return t[0:1] + t[1:2]


def _mlp_body(x_ref, a0_ref, a1_ref, selt_ref, side_ref, w1, b1, w2, b2, w3,
              b3, h_ref, sum_ref):
    i = pl.program_id(0)
    t = (x_ref[...] + (a0_ref[...] + a1_ref[...])
         + jnp.dot(selt_ref[...], side_ref[...],
                   preferred_element_type=jnp.float32, precision=_HI))
    h = jnp.maximum(jnp.dot(t, w1[...],
                            preferred_element_type=jnp.float32) + b1[...], 0.0)
    h = jnp.maximum(jnp.dot(h, w2[...],
                            preferred_element_type=jnp.float32) + b2[...], 0.0)
    h = jnp.maximum(jnp.dot(h, w3[...],
                            preferred_element_type=jnp.float32) + b3[...], 0.0)
    h_ref[...] = h

    def body(j, acc):
        return acc + h_ref[pl.ds(j * 8, 8), :]

    part = lax.fori_loop(0, _BLK // 8, body, jnp.zeros((8, _D), jnp.float32))

    @pl.when(i == 0)
    def _():
        sum_ref[...] = part

    @pl.when(i != 0)
    def _():
        sum_ref[...] = sum_ref[...] + part


def _var_body(h_ref, sum_ref, var_ref):
    i = pl.program_id(0)
    mean = _tree8(sum_ref[...]) / jnp.float32(_N)

    def body(j, acc):
        blk = h_ref[pl.ds(j * 8, 8), :] - mean
        return acc + blk * blk

    part = lax.fori_loop(0, _BLK // 8, body, jnp.zeros((8, _D), jnp.float32))

    @pl.when(i == 0)
    def _():
        var_ref[...] = part

    @pl.when(i != 0)
    def _():
        var_ref[...] = var_ref[...] + part


def _bn_body(h_ref, sum_ref, var_ref, g_ref, b_ref, out_ref, *, apply_relu):
    mean = _tree8(sum_ref[...]) / jnp.float32(_N)
    var = _tree8(var_ref[...]) / jnp.float32(_N)
    inv = g_ref[...] * lax.rsqrt(var + 1e-5)
    o = (h_ref[...] - mean) * inv + b_ref[...]
    if apply_relu:
        o = jnp.maximum(o, 0.0)
    out_ref[...] = o


_row_spec = pl.BlockSpec((_BLK, _D), lambda i: (i, 0))
_acc_spec = pl.BlockSpec((8, _D), lambda i: (0, 0))


def _enc_call(feat, depth2d, wenc, benc2d, demb):
    emb = pl.pallas_call(
        _emb_body,
        grid=(_NBLK,),
        in_specs=[
            pl.BlockSpec((_BLK, 1), lambda i: (i, 0)),
            pl.BlockSpec((_VOCAB, _D), lambda i: (0, 0)),
        ],
        out_specs=_row_spec,
        out_shape=jax.ShapeDtypeStruct((_N, _D), jnp.float32),
    )(depth2d, demb)
    return pl.pallas_call(
        _enc_body,
        grid=(_NBLK,),
        in_specs=[
            _row_spec,
            pl.BlockSpec((_D, _D), lambda i: (0, 0)),
            pl.BlockSpec((1, _D), lambda i: (0, 0)),
            _row_spec,
        ],
        out_specs=_row_spec,
        out_shape=jax.ShapeDtypeStruct((_N, _D), jnp.float32),
    )(feat, wenc, benc2d, emb)


def _mlp_call(x, a0, a1, selT, side, w1, b1, w2, b2, w3, b3):
    wspec = pl.BlockSpec((_D, _D), lambda i: (0, 0))
    bspec = pl.BlockSpec((1, _D), lambda i: (0, 0))
    return pl.pallas_call(
        _mlp_body,
        grid=(_NBLK,),
        in_specs=[_row_spec, _row_spec, _row_spec,
                  pl.BlockSpec((_BLK, 2 * _NW), lambda i: (i, 0)),
                  pl.BlockSpec((2 * _NW, _D), lambda i: (0, 0)),
                  wspec, bspec, wspec, bspec, wspec, bspec],
        out_specs=[_row_spec, _acc_spec],
        out_shape=[jax.ShapeDtypeStruct((_N, _D), jnp.float32),
                   jax.ShapeDtypeStruct((8, _D), jnp.float32)],
    )(x, a0, a1, selT, side, w1, b1, w2, b2, w3, b3)


def _var_call(h, hsum):
    return pl.pallas_call(
        _var_body,
        grid=(_NBLK,),
        in_specs=[_row_spec, _acc_spec],
        out_specs=_acc_spec,
        out_shape=jax.ShapeDtypeStruct((8, _D), jnp.float32),
    )(h, hsum)


def _bn_call(h, hsum, hvar, g, b, apply_relu):
    return pl.pallas_call(
        functools.partial(_bn_body, apply_relu=apply_relu),
        grid=(_NBLK,),
        in_specs=[_row_spec, _acc_spec, _acc_spec,
                  pl.BlockSpec((1, _D), lambda i: (0, 0)),
                  pl.BlockSpec((1, _D), lambda i: (0, 0))],
        out_specs=_row_spec,
        out_shape=jax.ShapeDtypeStruct((_N, _D), jnp.float32),
    )(h, hsum, hvar, g, b)


def kernel(feat, depth, edge_index, W_enc, b_enc, depth_emb,
           W1, b1, W2, b2, W3, b3, gamma, beta):
    src = edge_index[0]
    dst = edge_index[1]
    so, mflag, scat, fin, selT = _edge_metadata(src, dst)
    zeros = jnp.zeros((_NOUT, _D), jnp.float32)

    x = _enc_call(feat, depth.reshape(_N, 1), W_enc,
                  b_enc.reshape(1, _D), depth_emb)

    b1r = b1.reshape(1, _D)
    b2r = b2.reshape(1, _D)
    b3r = b3.reshape(1, _D)
    for l in range(_L):
        out0, out1 = _sc_agg(x, so, mflag, scat, fin, zeros)
        side = jnp.concatenate([out0[_N + _NDUMP:], out1[_N + _NDUMP:]], 0)
        h, hsum = _mlp_call(x, out0[:_N], out1[:_N], selT, side,
                            W1, b1r, W2, b2r, W3, b3r)
        hvar = _var_call(h, hsum)
        x = _bn_call(h, hsum, hvar, gamma[l].reshape(1, _D),
                     beta[l].reshape(1, _D), apply_relu=(l < _L - 1))
    return x
